# baseline (speedup 1.0000x reference)
import jax
import jax.numpy as jnp
from jax import lax
from jax.experimental import pallas as pl
from jax.experimental.pallas import tpu as pltpu
from jax.experimental.pallas import tpu_sc as plsc

B, D, T, K = 16, 64, 1024, 1024
N = B * T
BH = B // 2
NH = BH * T

_info = plsc.get_sparse_core_info()
_NW = _info.num_cores * _info.num_subcores
_BPW = NH // _NW
_CH = 128
_NCH = _BPW // _CH


def _argmin_body(x_ref, e_ref, idx_ref):
    xb = x_ref[0]
    e = e_ref[...]
    scores = lax.dot_general(
        xb, e, (((0,), (0,)), ((), ())), preferred_element_type=jnp.float32
    )
    x_sq = jnp.sum(xb * xb, axis=0)[:, None]
    e_sq = jnp.sum(e * e, axis=0)[None, :]
    d = x_sq - 2.0 * scores + e_sq
    m = jnp.min(d, axis=1, keepdims=True)
    fi = lax.broadcasted_iota(jnp.int32, (T, K), 1).astype(jnp.float32)
    fidx = jnp.min(jnp.where(d <= m, fi, float(K)), axis=1)
    idx_ref[...] = fidx.astype(jnp.int32).reshape(T // 128, 128)


def _gather_body(table_hbm, idx_hbm, out_hbm, idx_v, rows_v, sem):
    wid = lax.axis_index("s") * _info.num_cores + lax.axis_index("c")
    base = wid * _BPW
    pltpu.sync_copy(idx_hbm.at[pl.ds(wid * _NCH, _NCH)], idx_v)
    copies = [
        pltpu.async_copy(
            table_hbm.at[idx_v.at[j]], rows_v.at[pl.ds(j * _CH, _CH)], sem
        )
        for j in range(_NCH)
    ]
    for c in copies:
        c.wait()
    pltpu.sync_copy(rows_v, out_hbm.at[pl.ds(base, _BPW)])


def _finish_body(x_ref, q_ref, out_ref, loss_ref):
    b = pl.program_id(0)
    xb = x_ref[0]
    q = q_ref[0].T
    out_ref[0] = xb + (q - xb)
    diff = xb - q

    @pl.when(b == 0)
    def _():
        loss_ref[...] = jnp.zeros((1, 1), jnp.float32)

    loss_ref[...] += jnp.sum(diff * diff).reshape(1, 1)


def _finish_body_aliased(x_ref, q_ref, prev_ref, out_ref, loss_ref):
    del prev_ref
    _finish_body(x_ref, q_ref, out_ref, loss_ref)


def _stage1(x_in, e_i_ts, h):
    return pl.pallas_call(
        _argmin_body,
        grid=(BH,),
        in_specs=[
            pl.BlockSpec((1, D, T), lambda b: (b + h * BH, 0, 0)),
            pl.BlockSpec((D, K), lambda b: (0, 0)),
        ],
        out_specs=pl.BlockSpec((T // 128, 128), lambda b: (b, 0)),
        out_shape=jax.ShapeDtypeStruct((NH // 128, 128), jnp.int32),
    )(x_in, e_i_ts)


def _sc_gather(table, idx):
    gather = pl.kernel(
        _gather_body,
        mesh=plsc.VectorSubcoreMesh(core_axis_name="c", subcore_axis_name="s"),
        out_type=jax.ShapeDtypeStruct((NH, D), jnp.float32),
        scratch_types=[
            pltpu.VMEM((_NCH, _CH), jnp.int32),
            pltpu.VMEM((_BPW, D), jnp.float32),
            pltpu.SemaphoreType.DMA,
        ],
        compiler_params=pltpu.CompilerParams(use_tc_tiling_on_sc=False),
    )
    return gather(table, idx)


@jax.jit
def kernel(x_in, e_i_ts):
    table = e_i_ts.T
    qrows = []
    for h in range(2):
        idx_h = _stage1(x_in, e_i_ts, h)
        qrows.append(_sc_gather(table, idx_h).reshape(BH, T, D))

    q_a, loss_a = pl.pallas_call(
        _finish_body,
        grid=(BH,),
        in_specs=[
            pl.BlockSpec((1, D, T), lambda b: (b, 0, 0)),
            pl.BlockSpec((1, T, D), lambda b: (b, 0, 0)),
        ],
        out_specs=[
            pl.BlockSpec((1, D, T), lambda b: (b, 0, 0)),
            pl.BlockSpec((1, 1), lambda b: (0, 0)),
        ],
        out_shape=[
            jax.ShapeDtypeStruct((B, D, T), jnp.float32),
            jax.ShapeDtypeStruct((1, 1), jnp.float32),
        ],
        compiler_params=pltpu.CompilerParams(
            dimension_semantics=("arbitrary",),
        ),
    )(x_in, qrows[0])

    q_out, loss_b = pl.pallas_call(
        _finish_body_aliased,
        grid=(BH,),
        in_specs=[
            pl.BlockSpec((1, D, T), lambda b: (b + BH, 0, 0)),
            pl.BlockSpec((1, T, D), lambda b: (b, 0, 0)),
            pl.BlockSpec(memory_space=pl.ANY),
        ],
        out_specs=[
            pl.BlockSpec((1, D, T), lambda b: (b + BH, 0, 0)),
            pl.BlockSpec((1, 1), lambda b: (0, 0)),
        ],
        out_shape=[
            jax.ShapeDtypeStruct((B, D, T), jnp.float32),
            jax.ShapeDtypeStruct((1, 1), jnp.float32),
        ],
        input_output_aliases={2: 0},
        compiler_params=pltpu.CompilerParams(
            dimension_semantics=("arbitrary",),
        ),
    )(x_in, qrows[1], q_a)

    loss = (loss_a[0, 0] + loss_b[0, 0]) / (B * D * T)
    return (q_out, loss, loss)
